# baseline (speedup 1.0000x reference)
import jax
import jax.numpy as jnp
from jax import lax
from jax.experimental import pallas as pl
from jax.experimental.pallas import tpu as pltpu
from jax.experimental.pallas import tpu_sc as plsc

_ROW = 128
_V = 16


def _sc_degree(dst1d, zeros_n, *, n_acc, epw, nc, ns):
    chunks = epw // _ROW

    def body(dst_hbm, zeros_hbm, out_hbm, dst_v, priv):
        cid = lax.axis_index("c")
        sid = lax.axis_index("s")
        wid = cid * ns + sid
        pltpu.sync_copy(zeros_hbm, priv)
        e0 = wid * epw
        ones = jnp.full((_V,), 1.0, jnp.float32)

        def chunk(i, carry):
            pltpu.sync_copy(dst_hbm.at[pl.ds(e0 + i * _ROW, _ROW)], dst_v)
            for j in range(_ROW // _V):
                ivec = dst_v[pl.ds(j * _V, _V)]
                plsc.addupdate_scatter(priv, [ivec], ones)
            return carry

        lax.fori_loop(0, chunks, chunk, 0)
        pltpu.sync_copy(priv, out_hbm.at[pl.ds(wid * n_acc, n_acc)])

    return pl.kernel(
        body,
        out_type=jax.ShapeDtypeStruct((2 * ns * n_acc,), jnp.float32),
        mesh=plsc.VectorSubcoreMesh(core_axis_name="c", subcore_axis_name="s"),
        compiler_params=pltpu.CompilerParams(needs_layout_passes=False),
        scratch_types=[
            pltpu.VMEM((_ROW,), jnp.int32),
            pltpu.VMEM((n_acc,), jnp.float32),
        ],
    )(dst1d, zeros_n)


def _sc_segsum(xws, src1d, dst1d, zerosnd, *, n_acc, d, epw0, epw1, nc, ns):
    zrows = n_acc // ns

    assert epw0 % (2 * _ROW) == 0 and epw1 % (2 * _ROW) == 0

    def body(xws_hbm, src_hbm, dst_hbm, zeros_hbm, out_hbm,
             src_v0, dst_v0, rows_v0, src_v1, dst_v1, rows_v1,
             acc_sh, sem0, sem1):
        cid = lax.axis_index("c")
        sid = lax.axis_index("s")
        wid = cid * ns + sid
        pltpu.sync_copy(zeros_hbm.at[pl.ds(sid * zrows, zrows)],
                        acc_sh.at[pl.ds(sid * zrows, zrows)])
        plsc.subcore_barrier()
        e0 = jnp.where(cid == 0, sid * epw0, ns * epw0 + sid * epw1)
        chunks = jnp.where(cid == 0, epw0 // _ROW, epw1 // _ROW)

        pltpu.sync_copy(src_hbm.at[pl.ds(e0, _ROW)], src_v0)
        pltpu.sync_copy(dst_hbm.at[pl.ds(e0, _ROW)], dst_v0)
        pltpu.async_copy(xws_hbm.at[src_v0], rows_v0, sem0)

        def pair(o, carry):
            i0 = o * 2
            pltpu.sync_copy(src_hbm.at[pl.ds(e0 + (i0 + 1) * _ROW, _ROW)],
                            src_v1)
            pltpu.sync_copy(dst_hbm.at[pl.ds(e0 + (i0 + 1) * _ROW, _ROW)],
                            dst_v1)
            pltpu.async_copy(xws_hbm.at[src_v1], rows_v1, sem1)
            pltpu.make_async_copy(xws_hbm.at[src_v0], rows_v0, sem0).wait()
            pltpu.sync_copy(rows_v0, acc_sh.at[dst_v0], add=True)

            @pl.when(i0 + 2 < chunks)
            def _():
                pltpu.sync_copy(
                    src_hbm.at[pl.ds(e0 + (i0 + 2) * _ROW, _ROW)], src_v0)
                pltpu.sync_copy(
                    dst_hbm.at[pl.ds(e0 + (i0 + 2) * _ROW, _ROW)], dst_v0)
                pltpu.async_copy(xws_hbm.at[src_v0], rows_v0, sem0)

            pltpu.make_async_copy(xws_hbm.at[src_v1], rows_v1, sem1).wait()
            pltpu.sync_copy(rows_v1, acc_sh.at[dst_v1], add=True)
            return carry

        lax.fori_loop(0, chunks // 2, pair, 0, unroll=False)
        plsc.subcore_barrier()
        pltpu.sync_copy(acc_sh.at[pl.ds(sid * zrows, zrows)],
                        out_hbm.at[pl.ds(cid * n_acc + sid * zrows, zrows)])

    return pl.kernel(
        body,
        out_type=jax.ShapeDtypeStruct((2 * n_acc, d), jnp.float32),
        mesh=plsc.VectorSubcoreMesh(core_axis_name="c", subcore_axis_name="s"),
        scratch_types=[
            pltpu.VMEM((_ROW,), jnp.int32),
            pltpu.VMEM((_ROW,), jnp.int32),
            pltpu.VMEM((_ROW, d), jnp.float32),
            pltpu.VMEM((_ROW,), jnp.int32),
            pltpu.VMEM((_ROW,), jnp.int32),
            pltpu.VMEM((_ROW, d), jnp.float32),
            pltpu.VMEM_SHARED((n_acc, d), jnp.float32),
            pltpu.SemaphoreType.DMA,
            pltpu.SemaphoreType.DMA,
        ],
    )(xws, src1d, dst1d, zerosnd)


def _tc_xws_body(x_ref, w_ref, dg_ref, xws_ref, dinv_ref):
    deg = jnp.sum(dg_ref[...], axis=1, keepdims=True) + 1.0
    dinv = lax.rsqrt(deg)
    xw = jnp.dot(x_ref[...], w_ref[...], preferred_element_type=jnp.float32)
    xws_ref[...] = xw * dinv
    dinv_ref[...] = dinv


def _tc_final_body(a0_ref, a1_ref, xws_ref, dinv_ref, b_ref, out1_ref, h_ref):
    h = dinv_ref[...] * (a0_ref[...] + a1_ref[...] + xws_ref[...]) + b_ref[...]
    m = jnp.max(h, axis=1, keepdims=True)
    lse = jnp.log(jnp.sum(jnp.exp(h - m), axis=1, keepdims=True)) + m
    out1_ref[...] = h - lse
    h_ref[...] = h


def kernel(x, edge_index, W, b):
    n, d_in = x.shape
    d = W.shape[1]
    e = edge_index.shape[1]
    info = plsc.get_sparse_core_info()
    nc, ns = info.num_cores, info.num_subcores
    nw = nc * ns

    step = 2 * _ROW
    epw = -(-e // (nw * step)) * step
    e_pad = epw * nw
    src = edge_index[0].astype(jnp.int32)
    dst = edge_index[1].astype(jnp.int32)
    pad = e_pad - e
    n_acc = -(-(n + 1) // 128) * 128
    if pad:
        src = jnp.concatenate([src, jnp.zeros((pad,), jnp.int32)])
        dummy = n + (jnp.arange(pad, dtype=jnp.int32) % (n_acc - n))
        dst = jnp.concatenate([dst, dummy])

    deg_out = _sc_degree(
        dst, jnp.zeros((n_acc,), jnp.float32),
        n_acc=n_acc, epw=epw, nc=nc, ns=ns)
    degt = deg_out.reshape(nw, n_acc)[:, :n].T

    bn = 1000
    grid = (n // bn,)
    xws, dinv = pl.pallas_call(
        _tc_xws_body,
        grid=grid,
        in_specs=[
            pl.BlockSpec((bn, d_in), lambda i: (i, 0)),
            pl.BlockSpec((d_in, d), lambda i: (0, 0)),
            pl.BlockSpec((bn, nw), lambda i: (i, 0)),
        ],
        out_specs=[pl.BlockSpec((bn, d), lambda i: (i, 0)),
                   pl.BlockSpec((bn, 1), lambda i: (i, 0))],
        out_shape=[jax.ShapeDtypeStruct((n, d), jnp.float32),
                   jax.ShapeDtypeStruct((n, 1), jnp.float32)],
    )(x, W, degt)

    per_core = epw * 2
    epw0 = (per_core * 3 // 8) // (2 * _ROW) * (2 * _ROW)
    epw1 = per_core - epw0
    accp = _sc_segsum(
        xws, src, dst,
        jnp.zeros((n_acc, d), jnp.float32),
        n_acc=n_acc, d=d, epw0=epw0, epw1=epw1, nc=nc, ns=ns)

    out1, h = pl.pallas_call(
        _tc_final_body,
        grid=grid,
        in_specs=[
            pl.BlockSpec((bn, d), lambda i: (i, 0)),
            pl.BlockSpec((bn, d), lambda i: (i, 0)),
            pl.BlockSpec((bn, d), lambda i: (i, 0)),
            pl.BlockSpec((bn, 1), lambda i: (i, 0)),
            pl.BlockSpec((1, d), lambda i: (0, 0)),
        ],
        out_specs=[pl.BlockSpec((bn, d), lambda i: (i, 0)),
                   pl.BlockSpec((bn, d), lambda i: (i, 0))],
        out_shape=[jax.ShapeDtypeStruct((n, d), jnp.float32),
                   jax.ShapeDtypeStruct((n, d), jnp.float32)],
    )(accp[:n], accp[n_acc:n_acc + n], xws, dinv, b.reshape(1, d))

    return (out1, h)


# speedup vs baseline: 1.0640x; 1.0640x over previous
import jax
import jax.numpy as jnp
from jax import lax
from jax.experimental import pallas as pl
from jax.experimental.pallas import tpu as pltpu
from jax.experimental.pallas import tpu_sc as plsc

_ROW = 128
_V = 16


def _sc_degree(dst1d, zeros_n, *, n_acc, epw, nc, ns):
    chunks = epw // _ROW

    def body(dst_hbm, zeros_hbm, out_hbm, dst_v, priv):
        cid = lax.axis_index("c")
        sid = lax.axis_index("s")
        wid = cid * ns + sid
        pltpu.sync_copy(zeros_hbm, priv)
        e0 = wid * epw
        ones = jnp.full((_V,), 1.0, jnp.float32)

        def chunk(i, carry):
            pltpu.sync_copy(dst_hbm.at[pl.ds(e0 + i * _ROW, _ROW)], dst_v)
            for j in range(_ROW // _V):
                ivec = dst_v[pl.ds(j * _V, _V)]
                plsc.addupdate_scatter(priv, [ivec], ones)
            return carry

        lax.fori_loop(0, chunks, chunk, 0)
        pltpu.sync_copy(priv, out_hbm.at[pl.ds(wid * n_acc, n_acc)])

    return pl.kernel(
        body,
        out_type=jax.ShapeDtypeStruct((2 * ns * n_acc,), jnp.float32),
        mesh=plsc.VectorSubcoreMesh(core_axis_name="c", subcore_axis_name="s"),
        compiler_params=pltpu.CompilerParams(needs_layout_passes=False),
        scratch_types=[
            pltpu.VMEM((_ROW,), jnp.int32),
            pltpu.VMEM((n_acc,), jnp.float32),
        ],
    )(dst1d, zeros_n)


def _sc_segsum(xws, src1d, dst1d, zerosnd, *, n_acc, d, epw0, epw1, nc, ns):
    zrows = n_acc // ns

    assert epw0 % (2 * _ROW) == 0 and epw1 % (2 * _ROW) == 0

    def body(xws_hbm, src_hbm, dst_hbm, zeros_hbm, out_hbm,
             src_v0, dst_v0, rows_v0, src_v1, dst_v1, rows_v1,
             acc_sh, sem0, sem1):
        cid = lax.axis_index("c")
        sid = lax.axis_index("s")
        wid = cid * ns + sid
        pltpu.sync_copy(zeros_hbm.at[pl.ds(sid * zrows, zrows)],
                        acc_sh.at[pl.ds(sid * zrows, zrows)])
        plsc.subcore_barrier()
        e0 = jnp.where(cid == 0, sid * epw0, ns * epw0 + sid * epw1)
        chunks = jnp.where(cid == 0, epw0 // _ROW, epw1 // _ROW)

        pltpu.sync_copy(src_hbm.at[pl.ds(e0, _ROW)], src_v0)
        pltpu.sync_copy(dst_hbm.at[pl.ds(e0, _ROW)], dst_v0)
        pltpu.async_copy(xws_hbm.at[src_v0], rows_v0, sem0)

        def pair(o, carry):
            i0 = o * 2
            pltpu.sync_copy(src_hbm.at[pl.ds(e0 + (i0 + 1) * _ROW, _ROW)],
                            src_v1)
            pltpu.sync_copy(dst_hbm.at[pl.ds(e0 + (i0 + 1) * _ROW, _ROW)],
                            dst_v1)
            pltpu.async_copy(xws_hbm.at[src_v1], rows_v1, sem1)
            pltpu.make_async_copy(xws_hbm.at[src_v0], rows_v0, sem0).wait()
            pltpu.sync_copy(rows_v0, acc_sh.at[dst_v0], add=True)

            @pl.when(i0 + 2 < chunks)
            def _():
                pltpu.sync_copy(
                    src_hbm.at[pl.ds(e0 + (i0 + 2) * _ROW, _ROW)], src_v0)
                pltpu.sync_copy(
                    dst_hbm.at[pl.ds(e0 + (i0 + 2) * _ROW, _ROW)], dst_v0)
                pltpu.async_copy(xws_hbm.at[src_v0], rows_v0, sem0)

            pltpu.make_async_copy(xws_hbm.at[src_v1], rows_v1, sem1).wait()
            pltpu.sync_copy(rows_v1, acc_sh.at[dst_v1], add=True)
            return carry

        lax.fori_loop(0, chunks // 2, pair, 0, unroll=False)
        plsc.subcore_barrier()
        pltpu.sync_copy(acc_sh.at[pl.ds(sid * zrows, zrows)],
                        out_hbm.at[pl.ds(cid * n_acc + sid * zrows, zrows)])

    return pl.kernel(
        body,
        out_type=jax.ShapeDtypeStruct((2 * n_acc, d), jnp.float32),
        mesh=plsc.VectorSubcoreMesh(core_axis_name="c", subcore_axis_name="s"),
        scratch_types=[
            pltpu.VMEM((_ROW,), jnp.int32),
            pltpu.VMEM((_ROW,), jnp.int32),
            pltpu.VMEM((_ROW, d), jnp.float32),
            pltpu.VMEM((_ROW,), jnp.int32),
            pltpu.VMEM((_ROW,), jnp.int32),
            pltpu.VMEM((_ROW, d), jnp.float32),
            pltpu.VMEM_SHARED((n_acc, d), jnp.float32),
            pltpu.SemaphoreType.DMA,
            pltpu.SemaphoreType.DMA,
        ],
    )(xws, src1d, dst1d, zerosnd)


def _tc_xws_body(x_ref, w_ref, dg_ref, xws_ref, dinv_ref):
    deg = jnp.sum(dg_ref[...], axis=1, keepdims=True) + 1.0
    dinv = lax.rsqrt(deg)
    xw = jnp.dot(x_ref[...], w_ref[...], preferred_element_type=jnp.float32)
    xws_ref[...] = xw * dinv
    dinv_ref[...] = dinv


def _tc_final_body(a0_ref, a1_ref, xws_ref, dinv_ref, b_ref, out1_ref, h_ref):
    h = dinv_ref[...] * (a0_ref[...] + a1_ref[...] + xws_ref[...]) + b_ref[...]
    m = jnp.max(h, axis=1, keepdims=True)
    lse = jnp.log(jnp.sum(jnp.exp(h - m), axis=1, keepdims=True)) + m
    out1_ref[...] = h - lse
    h_ref[...] = h


def kernel(x, edge_index, W, b):
    n, d_in = x.shape
    d = W.shape[1]
    e = edge_index.shape[1]
    info = plsc.get_sparse_core_info()
    nc, ns = info.num_cores, info.num_subcores
    nw = nc * ns

    step = 2 * _ROW
    epw = -(-e // (nw * step)) * step
    e_pad = epw * nw
    src = edge_index[0].astype(jnp.int32)
    dst = edge_index[1].astype(jnp.int32)
    pad = e_pad - e
    n_acc = -(-(n + 1) // 128) * 128
    if pad:
        src = jnp.concatenate([src, jnp.zeros((pad,), jnp.int32)])
        dummy = n + (jnp.arange(pad, dtype=jnp.int32) % (n_acc - n))
        dst = jnp.concatenate([dst, dummy])

    deg_out = _sc_degree(
        dst, jnp.zeros((n_acc,), jnp.float32),
        n_acc=n_acc, epw=epw, nc=nc, ns=ns)
    degt = deg_out.reshape(nw, n_acc)[:, :n].T

    bn = 1000
    grid = (n // bn,)
    xws, dinv = pl.pallas_call(
        _tc_xws_body,
        grid=grid,
        in_specs=[
            pl.BlockSpec((bn, d_in), lambda i: (i, 0)),
            pl.BlockSpec((d_in, d), lambda i: (0, 0)),
            pl.BlockSpec((bn, nw), lambda i: (i, 0)),
        ],
        out_specs=[pl.BlockSpec((bn, d), lambda i: (i, 0)),
                   pl.BlockSpec((bn, 1), lambda i: (i, 0))],
        out_shape=[jax.ShapeDtypeStruct((n, d), jnp.float32),
                   jax.ShapeDtypeStruct((n, 1), jnp.float32)],
    )(x, W, degt)

    per_core = epw * 2
    epw1 = (per_core * 3 // 8) // (2 * _ROW) * (2 * _ROW)
    epw0 = per_core - epw1
    accp = _sc_segsum(
        xws, src, dst,
        jnp.zeros((n_acc, d), jnp.float32),
        n_acc=n_acc, d=d, epw0=epw0, epw1=epw1, nc=nc, ns=ns)

    out1, h = pl.pallas_call(
        _tc_final_body,
        grid=grid,
        in_specs=[
            pl.BlockSpec((bn, d), lambda i: (i, 0)),
            pl.BlockSpec((bn, d), lambda i: (i, 0)),
            pl.BlockSpec((bn, d), lambda i: (i, 0)),
            pl.BlockSpec((bn, 1), lambda i: (i, 0)),
            pl.BlockSpec((1, d), lambda i: (0, 0)),
        ],
        out_specs=[pl.BlockSpec((bn, d), lambda i: (i, 0)),
                   pl.BlockSpec((bn, d), lambda i: (i, 0))],
        out_shape=[jax.ShapeDtypeStruct((n, d), jnp.float32),
                   jax.ShapeDtypeStruct((n, d), jnp.float32)],
    )(accp[:n], accp[n_acc:n_acc + n], xws, dinv, b.reshape(1, d))

    return (out1, h)
